# Initial kernel scaffold; baseline (speedup 1.0000x reference)
import jax
import jax.numpy as jnp
from jax import lax
from jax.experimental import pallas as pl
from jax.experimental.pallas import tpu as pltpu
from jax.experimental.pallas import tpu_sc as plsc

NC = 2
NS = 16
NW = NC * NS
LANES = 16
K = 128


def _sc_mesh():
    return plsc.VectorSubcoreMesh(core_axis_name="c", subcore_axis_name="s")


def _make_sc_agg(N_sp, D, C):
    RPT = N_sp // NS
    G = C // 2

    def body(h_hbm, src_hbm, dst_hbm, z2_hbm, out_hbm,
             src_v, dst_v, rows0, rows1, agg_sh, sem0, sem1):
        c = lax.axis_index("c")
        s = lax.axis_index("s")
        w = c * NS + s

        r0 = s * RPT
        pltpu.sync_copy(z2_hbm.at[pl.ds(r0, RPT)], agg_sh.at[pl.ds(r0, RPT)])

        pltpu.sync_copy(src_hbm.at[w], src_v)
        pltpu.sync_copy(dst_hbm.at[w], dst_v)
        plsc.subcore_barrier()

        def step(g, carry):
            a = 2 * g
            b = a + 1
            d0 = pltpu.async_copy(h_hbm.at[src_v.at[a]], rows0, sem0)
            d1 = pltpu.async_copy(h_hbm.at[src_v.at[b]], rows1, sem1)
            d0.wait()
            pltpu.sync_copy(rows0, agg_sh.at[dst_v.at[a]], add=True)
            d1.wait()
            pltpu.sync_copy(rows1, agg_sh.at[dst_v.at[b]], add=True)
            return carry

        lax.fori_loop(0, G, step, 0)
        plsc.subcore_barrier()

        pltpu.sync_copy(agg_sh.at[pl.ds(r0, RPT)],
                        out_hbm.at[c, pl.ds(r0, RPT)])

    def call(h, src_r, dst_r, z2):
        kern = pl.kernel(
            body,
            out_type=jax.ShapeDtypeStruct((NC, N_sp, D), jnp.float32),
            mesh=_sc_mesh(),
            scratch_types=[
                pltpu.VMEM((C, K), jnp.int32),
                pltpu.VMEM((C, K), jnp.int32),
                pltpu.VMEM((K, D), jnp.float32),
                pltpu.VMEM((K, D), jnp.float32),
                pltpu.VMEM_SHARED((N_sp, D), jnp.float32),
                pltpu.SemaphoreType.DMA,
                pltpu.SemaphoreType.DMA,
            ],
        )
        return kern(h, src_r, dst_r, z2)

    return call


def _make_sc_deg(N_sp, C):
    RPT = N_sp // NS
    G = C // 2

    def body(dst_hbm, z1_hbm, out_hbm, dst_v, ones_v, deg_sh):
        c = lax.axis_index("c")
        s = lax.axis_index("s")
        w = c * NS + s

        r0 = s * RPT
        pltpu.sync_copy(z1_hbm.at[pl.ds(r0, RPT)], deg_sh.at[pl.ds(r0, RPT)])
        pltpu.sync_copy(dst_hbm.at[w], dst_v)
        for i in range(K // LANES):
            ones_v[pl.ds(i * LANES, LANES)] = jnp.full(
                (LANES,), 1.0, jnp.float32)
        plsc.subcore_barrier()

        def step(g, carry):
            a = 2 * g
            pltpu.sync_copy(ones_v, deg_sh.at[dst_v.at[a]], add=True)
            pltpu.sync_copy(ones_v, deg_sh.at[dst_v.at[a + 1]], add=True)
            return carry

        lax.fori_loop(0, G, step, 0)
        plsc.subcore_barrier()
        pltpu.sync_copy(deg_sh.at[pl.ds(r0, RPT)],
                        out_hbm.at[c, pl.ds(r0, RPT)])

    def call(dst_r, z1):
        kern = pl.kernel(
            body,
            out_type=jax.ShapeDtypeStruct((NC, N_sp), jnp.float32),
            mesh=_sc_mesh(),
            scratch_types=[
                pltpu.VMEM((C, K), jnp.int32),
                pltpu.VMEM((K,), jnp.float32),
                pltpu.VMEM_SHARED((N_sp,), jnp.float32),
            ],
        )
        return kern(dst_r, z1)

    return call


def _tc_layer(p, degp, h, Wl, bl, Wr, relu):
    N, D = h.shape
    BM = 1000
    grid = (N // BM,)

    def body(p_ref, deg_ref, h_ref, wl_ref, bl_ref, wr_ref, o_ref):
        agg = p_ref[0] + p_ref[1]
        deg = deg_ref[0] + deg_ref[1]
        inv = 1.0 / jnp.maximum(deg, 1.0)
        z = (jnp.dot(agg, wl_ref[...], preferred_element_type=jnp.float32)
             * inv
             + bl_ref[...]
             + jnp.dot(h_ref[...], wr_ref[...],
                       preferred_element_type=jnp.float32))
        if relu:
            z = jnp.maximum(z, 0.0)
        o_ref[...] = z

    return pl.pallas_call(
        body,
        grid=grid,
        in_specs=[
            pl.BlockSpec((NC, BM, D), lambda i: (0, i, 0)),
            pl.BlockSpec((NC, BM, 1), lambda i: (0, i, 0)),
            pl.BlockSpec((BM, D), lambda i: (i, 0)),
            pl.BlockSpec((D, D), lambda i: (0, 0)),
            pl.BlockSpec((1, D), lambda i: (0, 0)),
            pl.BlockSpec((D, D), lambda i: (0, 0)),
        ],
        out_specs=pl.BlockSpec((BM, D), lambda i: (i, 0)),
        out_shape=jax.ShapeDtypeStruct((N, D), jnp.float32),
    )(p, degp, h, Wl, bl.reshape(1, D), Wr)


def kernel(x, edge_index, Wl0, bl0, Wr0, Wl1, bl1, Wr1, Wl2, bl2, Wr2):
    N, D = x.shape
    E = edge_index.shape[1]

    per_tile = -(-E // NW)
    C = -(-per_tile // K)
    C += C % 2
    E_pad = NW * C * K
    N_sp = -(-(N + 1) // 256) * 256

    src = edge_index[0]
    dst = edge_index[1]
    pad = E_pad - E
    src_r = jnp.concatenate(
        [src, jnp.zeros((pad,), jnp.int32)]).reshape(NW, C, K)
    dst_r = jnp.concatenate(
        [dst, jnp.full((pad,), N, jnp.int32)]).reshape(NW, C, K)
    z2 = jnp.zeros((N_sp, D), jnp.float32)
    z1 = jnp.zeros((N_sp,), jnp.float32)

    sc_agg = _make_sc_agg(N_sp, D, C)
    sc_deg = _make_sc_deg(N_sp, C)

    degp = sc_deg(dst_r, z1)[:, :N].reshape(NC, N, 1)

    h = x
    layers = [(Wl0, bl0, Wr0, True), (Wl1, bl1, Wr1, True),
              (Wl2, bl2, Wr2, False)]
    for Wl, bl, Wr, relu in layers:
        p = sc_agg(h, src_r, dst_r, z2)[:, :N, :]
        h = _tc_layer(p, degp, h, Wl, bl, Wr, relu)
    return h


# speedup vs baseline: 1.9544x; 1.9544x over previous
import jax
import jax.numpy as jnp
from jax import lax
from jax.experimental import pallas as pl
from jax.experimental.pallas import tpu as pltpu
from jax.experimental.pallas import tpu_sc as plsc

NC = 2
NS = 16
LANES = 16
K = 128


def _sc_mesh():
    return plsc.VectorSubcoreMesh(core_axis_name="c", subcore_axis_name="s")


def _make_sc_agg(H_sp, D, C):
    RPT = H_sp // NS
    G = C // 2

    def body(h_hbm, src_hbm, dstc_hbm, z2_hbm, out_hbm,
             src_v, dst_v, rows0, rows1, agg_sh, sem0, sem1):
        c = lax.axis_index("c")
        s = lax.axis_index("s")

        r0 = s * RPT
        pltpu.sync_copy(z2_hbm.at[pl.ds(r0, RPT)], agg_sh.at[pl.ds(r0, RPT)])

        pltpu.sync_copy(src_hbm.at[s], src_v)
        pltpu.sync_copy(dstc_hbm.at[c, s], dst_v)
        plsc.subcore_barrier()

        def step(g, carry):
            a = 2 * g
            b = a + 1
            d0 = pltpu.async_copy(h_hbm.at[src_v.at[a]], rows0, sem0)
            d1 = pltpu.async_copy(h_hbm.at[src_v.at[b]], rows1, sem1)
            d0.wait()
            pltpu.sync_copy(rows0, agg_sh.at[dst_v.at[a]], add=True)
            d1.wait()
            pltpu.sync_copy(rows1, agg_sh.at[dst_v.at[b]], add=True)
            return carry

        lax.fori_loop(0, G, step, 0)
        plsc.subcore_barrier()

        pltpu.sync_copy(agg_sh.at[pl.ds(r0, RPT)],
                        out_hbm.at[c, pl.ds(r0, RPT)])

    def call(h, src_r, dstc, z2):
        kern = pl.kernel(
            body,
            out_type=jax.ShapeDtypeStruct((NC, H_sp, D), jnp.float32),
            mesh=_sc_mesh(),
            scratch_types=[
                pltpu.VMEM((C, K), jnp.int32),
                pltpu.VMEM((C, K), jnp.int32),
                pltpu.VMEM((K, D), jnp.float32),
                pltpu.VMEM((K, D), jnp.float32),
                pltpu.VMEM_SHARED((H_sp, D), jnp.float32),
                pltpu.SemaphoreType.DMA,
                pltpu.SemaphoreType.DMA,
            ],
        )
        return kern(h, src_r, dstc, z2)

    return call


def _make_sc_deg(H_sp, C):
    RPT = H_sp // NS
    G = C // 2

    def body(dstc_hbm, out0_hbm, out1_hbm, dst_v, ones_v, dz_v, deg_sh):
        c = lax.axis_index("c")
        s = lax.axis_index("s")

        r0 = s * RPT
        for i in range(RPT // LANES):
            dz_v[pl.ds(i * LANES, LANES)] = jnp.zeros((LANES,), jnp.float32)
        pltpu.sync_copy(dz_v, deg_sh.at[pl.ds(r0, RPT)])
        pltpu.sync_copy(dstc_hbm.at[c, s], dst_v)
        for i in range(K // LANES):
            ones_v[pl.ds(i * LANES, LANES)] = jnp.full(
                (LANES,), 1.0, jnp.float32)
        plsc.subcore_barrier()

        def step(g, carry):
            a = 2 * g
            pltpu.sync_copy(ones_v, deg_sh.at[dst_v.at[a]], add=True)
            pltpu.sync_copy(ones_v, deg_sh.at[dst_v.at[a + 1]], add=True)
            return carry

        lax.fori_loop(0, G, step, 0)
        plsc.subcore_barrier()
        pltpu.sync_copy(deg_sh.at[pl.ds(r0, RPT)], dz_v)

        @pl.when(c == 0)
        def _():
            pltpu.sync_copy(dz_v, out0_hbm.at[pl.ds(r0, RPT)])

        @pl.when(c == 1)
        def _():
            pltpu.sync_copy(dz_v, out1_hbm.at[pl.ds(r0, RPT)])

    def call(dstc):
        kern = pl.kernel(
            body,
            out_type=[jax.ShapeDtypeStruct((H_sp,), jnp.float32),
                      jax.ShapeDtypeStruct((H_sp,), jnp.float32)],
            mesh=_sc_mesh(),
            scratch_types=[
                pltpu.VMEM((C, K), jnp.int32),
                pltpu.VMEM((K,), jnp.float32),
                pltpu.VMEM((RPT,), jnp.float32),
                pltpu.VMEM_SHARED((H_sp,), jnp.float32),
            ],
        )
        return kern(dstc)

    return call


def _tc_layer(p, degp, h, Wl, bl, Wr, relu):
    N, D = h.shape
    BM = 1000
    H = N // NC
    KB = H // BM
    grid = (N // BM,)

    def body(p_ref, deg_ref, h_ref, wl_ref, bl_ref, wr_ref, o_ref):
        agg = p_ref[0]
        deg = deg_ref[0]
        inv = 1.0 / jnp.maximum(deg, 1.0)
        z = (jnp.dot(agg, wl_ref[...], preferred_element_type=jnp.float32)
             * inv
             + bl_ref[...]
             + jnp.dot(h_ref[...], wr_ref[...],
                       preferred_element_type=jnp.float32))
        if relu:
            z = jnp.maximum(z, 0.0)
        o_ref[...] = z

    return pl.pallas_call(
        body,
        grid=grid,
        in_specs=[
            pl.BlockSpec((1, BM, D), lambda i: (i // KB, i % KB, 0)),
            pl.BlockSpec((1, BM, 1), lambda i: (i // KB, i % KB, 0)),
            pl.BlockSpec((BM, D), lambda i: (i, 0)),
            pl.BlockSpec((D, D), lambda i: (0, 0)),
            pl.BlockSpec((1, D), lambda i: (0, 0)),
            pl.BlockSpec((D, D), lambda i: (0, 0)),
        ],
        out_specs=pl.BlockSpec((BM, D), lambda i: (i, 0)),
        out_shape=jax.ShapeDtypeStruct((N, D), jnp.float32),
    )(p, degp, h, Wl, bl.reshape(1, D), Wr)


def kernel(x, edge_index, Wl0, bl0, Wr0, Wl1, bl1, Wr1, Wl2, bl2, Wr2):
    N, D = x.shape
    E = edge_index.shape[1]
    H = N // NC

    per_tile = -(-E // NS)
    C = -(-per_tile // K)
    C = -(-C // 8) * 8
    E_pad = NS * C * K
    H_sp = -(-(H + 1) // 256) * 256

    src = edge_index[0]
    dst = edge_index[1]
    pad = E_pad - E
    src_r = jnp.concatenate(
        [src, jnp.zeros((pad,), jnp.int32)]).reshape(NS, C, K)
    dst_p = jnp.concatenate([dst, jnp.full((pad,), N, jnp.int32)])
    halves = []
    for c in range(NC):
        lo = c * H
        d = dst_p - lo
        halves.append(jnp.where((d >= 0) & (d < H), d, H))
    dstc = jnp.stack(halves).reshape(NC, NS, C, K)
    z2 = jnp.zeros((H_sp, D), jnp.float32)

    sc_agg = _make_sc_agg(H_sp, D, C)
    sc_deg = _make_sc_deg(H_sp, C)

    deg0, deg1 = sc_deg(dstc)
    degp = jnp.stack([deg0[:H], deg1[:H]]).reshape(NC, H, 1)

    h = x
    layers = [(Wl0, bl0, Wr0, True), (Wl1, bl1, Wr1, True),
              (Wl2, bl2, Wr2, False)]
    for Wl, bl, Wr, relu in layers:
        p = sc_agg(h, src_r, dstc, z2)[:, :H, :]
        h = _tc_layer(p, degp, h, Wl, bl, Wr, relu)
    return h
